# Initial kernel scaffold; baseline (speedup 1.0000x reference)
import jax
import jax.numpy as jnp
from jax.experimental import pallas as pl


def kernel(h, edge_index, coord, edge_attr, We1, be1, We2, be2, Wa, ba, Wn1, bn1, Wn2, bn2, Wc1, bc1, Wc2):
    raise NotImplementedError("write your pallas kernel here")


# speedup vs baseline: 1.1232x; 1.1232x over previous
import functools

import jax
import jax.numpy as jnp
from jax.experimental import pallas as pl
from jax.experimental.pallas import tpu as pltpu

N = 10000
E = 320000
D = 128
H = 128
DE = 16

BN = 1000
BE = 640


def _silu(x):
    return x * jax.nn.sigmoid(x)


def _pre_kernel(h_ref, whr_ref, whc_ref, be1_ref, a_ref, b_ref):
    h = h_ref[...]
    a_ref[...] = h @ whr_ref[...] + be1_ref[...]
    b_ref[...] = h @ whc_ref[...]


def _precompute_ab(h, WhrT, WhcT, be1):
    return pl.pallas_call(
        _pre_kernel,
        grid=(N // BN,),
        in_specs=[
            pl.BlockSpec((BN, D), lambda i: (i, 0)),
            pl.BlockSpec((D, H), lambda i: (0, 0)),
            pl.BlockSpec((D, H), lambda i: (0, 0)),
            pl.BlockSpec((1, H), lambda i: (0, 0)),
        ],
        out_specs=[
            pl.BlockSpec((BN, H), lambda i: (i, 0)),
            pl.BlockSpec((BN, H), lambda i: (i, 0)),
        ],
        out_shape=[
            jax.ShapeDtypeStruct((N, H), jnp.float32),
            jax.ShapeDtypeStruct((N, H), jnp.float32),
        ],
    )(h, WhrT, WhcT, be1.reshape(1, H))


def _edge_kernel(g_ref, ea_ref, weaT_ref, we2T_ref, be2_ref, wa_ref, ba_ref,
                 wc1T_ref, bc1_ref, wc2_ref, ef_ref, cm_ref):
    t1 = g_ref[...] + ea_ref[...] @ weaT_ref[...]
    u = _silu(t1)
    v = _silu(u @ we2T_ref[...] + be2_ref[...])
    att = jax.nn.sigmoid(jnp.sum(v * wa_ref[...], axis=1, keepdims=True)
                         + ba_ref[0, 0])
    ef = v * att
    c1 = _silu(ef @ wc1T_ref[...] + bc1_ref[...])
    cm = jnp.sum(c1 * wc2_ref[...], axis=1)
    ef_ref[...] = ef
    cm_ref[...] = cm[None, :]


def _edge_mlp(G, edge_attr, WeaT, We2T, be2, Wa, ba, Wc1T, bc1, Wc2):
    return pl.pallas_call(
        _edge_kernel,
        grid=(E // BE,),
        in_specs=[
            pl.BlockSpec((BE, H), lambda i: (i, 0)),
            pl.BlockSpec((BE, DE), lambda i: (i, 0)),
            pl.BlockSpec((DE, H), lambda i: (0, 0)),
            pl.BlockSpec((H, H), lambda i: (0, 0)),
            pl.BlockSpec((1, H), lambda i: (0, 0)),
            pl.BlockSpec((1, H), lambda i: (0, 0)),
            pl.BlockSpec((1, 1), lambda i: (0, 0)),
            pl.BlockSpec((H, H), lambda i: (0, 0)),
            pl.BlockSpec((1, H), lambda i: (0, 0)),
            pl.BlockSpec((1, H), lambda i: (0, 0)),
        ],
        out_specs=[
            pl.BlockSpec((BE, H), lambda i: (i, 0)),
            pl.BlockSpec((1, BE), lambda i: (0, i)),
        ],
        out_shape=[
            jax.ShapeDtypeStruct((E, H), jnp.float32),
            jax.ShapeDtypeStruct((1, E), jnp.float32),
        ],
    )(G, edge_attr, WeaT, We2T, be2.reshape(1, H), Wa.reshape(1, H),
      ba.reshape(1, 1), Wc1T, bc1.reshape(1, H), Wc2.reshape(1, H))


def _node_kernel(h_ref, agg_ref, cacc_ref, wnhT_ref, wnaT_ref, bn1_ref,
                 wn2T_ref, bn2_ref, hout_ref, cout_ref):
    h = h_ref[...]
    u = _silu(h @ wnhT_ref[...] + agg_ref[...] @ wnaT_ref[...] + bn1_ref[...])
    hout_ref[...] = h + u @ wn2T_ref[...] + bn2_ref[...]
    cacc = cacc_ref[...]
    cnt = jnp.maximum(cacc[:, 3:4], 1.0)
    cout_ref[...] = cacc / cnt


def _node_model(h, agg, cacc, WnhT, WnaT, bn1, Wn2T, bn2):
    return pl.pallas_call(
        _node_kernel,
        grid=(N // BN,),
        in_specs=[
            pl.BlockSpec((BN, D), lambda i: (i, 0)),
            pl.BlockSpec((BN, H), lambda i: (i, 0)),
            pl.BlockSpec((BN, 16), lambda i: (i, 0)),
            pl.BlockSpec((D, H), lambda i: (0, 0)),
            pl.BlockSpec((H, H), lambda i: (0, 0)),
            pl.BlockSpec((1, H), lambda i: (0, 0)),
            pl.BlockSpec((H, D), lambda i: (0, 0)),
            pl.BlockSpec((1, D), lambda i: (0, 0)),
        ],
        out_specs=[
            pl.BlockSpec((BN, D), lambda i: (i, 0)),
            pl.BlockSpec((BN, 16), lambda i: (i, 0)),
        ],
        out_shape=[
            jax.ShapeDtypeStruct((N, D), jnp.float32),
            jax.ShapeDtypeStruct((N, 16), jnp.float32),
        ],
    )(h, agg, cacc, WnhT, WnaT, bn1.reshape(1, H), Wn2T, bn2.reshape(1, D))


def kernel(h, edge_index, coord, edge_attr, We1, be1, We2, be2, Wa, ba,
           Wn1, bn1, Wn2, bn2, Wc1, bc1, Wc2):
    row = edge_index[0]
    col = edge_index[1]
    WhrT = We1[:, :D].T
    WhcT = We1[:, D:2 * D].T
    w_r = We1[:, 2 * D]
    WeaT = We1[:, 2 * D + 1:].T

    A, B = _precompute_ab(h, WhrT, WhcT, be1)

    coord_diff = coord[row] - coord[col]
    radial = jnp.sum(coord_diff ** 2, axis=1, keepdims=True)
    G = A[row] + B[col] + radial * w_r[None, :]

    ef, cm = _edge_mlp(G, edge_attr, WeaT, We2T=We2.T, be2=be2, Wa=Wa, ba=ba,
                       Wc1T=Wc1.T, bc1=bc1, Wc2=Wc2)

    agg = jax.ops.segment_sum(ef, row, num_segments=N)
    trans = coord_diff * cm[0][:, None]
    pack = jnp.concatenate(
        [trans, jnp.ones((E, 1), jnp.float32),
         jnp.zeros((E, 12), jnp.float32)], axis=1)
    cacc = jax.ops.segment_sum(pack, row, num_segments=N)

    h_out, cupd = _node_model(h, agg, cacc, Wn1[:, :D].T, Wn1[:, D:].T,
                              bn1, Wn2.T, bn2)
    coord_out = coord + cupd[:, :3]
    return (h_out, coord_out, edge_attr)
